# baseline (speedup 1.0000x reference)
import jax
import jax.numpy as jnp
from jax import lax
from jax.experimental import pallas as pl
from jax.experimental.pallas import tpu as pltpu
from jax.experimental.pallas import tpu_sc as plsc

_B, _T, _V = 160, 16, 10000
_TAU = 0.13
_ALPHA = 0.7

_NC, _NS, _L = 2, 16, 16
_NW = _NC * _NS
_N = _B * _T
_TPW = _N // _NW
_K = 2
_NCHUNK = _TPW // _K
_NVEC = _V // _L


def _sc_body(inp2, inpflat, tgt, tgt_pad, maskv, sim, out,
             idx_v, idx2_v, flatidx_v, mask_v, mlvals_v, rows_v, inprows_v,
             stage_v, sem_r0, sem_r1, sem_i0, sem_i1, sem_ml, sem_out):
    wid = lax.axis_index("s") * _NC + lax.axis_index("c")
    base = wid * _TPW
    iota = lax.broadcasted_iota(jnp.int32, (_L,), 0)

    pltpu.sync_copy(tgt.at[pl.ds(base, _TPW)], idx_v)
    pltpu.sync_copy(tgt_pad.at[pl.ds(wid * _NCHUNK, _NCHUNK)], idx2_v)
    pltpu.sync_copy(maskv.at[pl.ds(base, _TPW)], mask_v)

    for k in range(_TPW // _L):
        sl = pl.ds(k * _L, _L)
        t16 = idx_v[sl]
        rowid = iota + (base + k * _L)
        flatidx_v[sl] = rowid * _V + t16
    ml_copy = pltpu.async_copy(inpflat.at[flatidx_v], mlvals_v, sem_ml)

    sem_r = (sem_r0, sem_r1)
    sem_i = (sem_i0, sem_i1)

    handles = {}

    def issue_tracked(g, b):
        h1 = pltpu.async_copy(sim.at[idx2_v.at[g, pl.ds(0, _K)]],
                              rows_v.at[b], sem_r[b])
        h2 = pltpu.async_copy(inp2.at[pl.ds(base + g * _K, _K)],
                              inprows_v.at[b], sem_i[b])
        handles[g] = (h1, h2)

    issue_tracked(0, 0)
    issue_tracked(1, 1)

    zeros = jnp.zeros((_L,), jnp.float32)
    inv_tau = jnp.float32(1.0 / _TAU)
    smooth16 = zeros
    dacc = zeros
    sacc = zeros
    chunks_per_group = _L // _K

    for g in range(_NCHUNK):
        b = g % 2
        h1, h2 = handles.pop(g)
        h1.wait()
        h2.wait()
        for tok in range(_K):
            def body(j, carry):
                dv, sv = carry
                sl = pl.ds(j * _L, _L)
                e = jnp.exp(rows_v[b, tok, sl] * inv_tau)
                return (dv + e * inprows_v[b, tok, sl], sv + e)

            dv, sv = lax.fori_loop(0, _NVEC, body, (zeros, zeros), unroll=4)
            tokidx = g * _K + tok
            lane = tokidx % _L
            dacc = dacc + jnp.where(iota == lane, jnp.sum(dv), 0.0)
            sacc = sacc + jnp.where(iota == lane, jnp.sum(sv), 0.0)
        if g % chunks_per_group == chunks_per_group - 1:
            grp = g // chunks_per_group
            m16 = mask_v[pl.ds(grp * _L, _L)]
            smooth16 = smooth16 + m16 * dacc / sacc
            dacc = zeros
            sacc = zeros
        if g + 2 < _NCHUNK:
            issue_tracked(g + 2, b)
    smooth_acc = jnp.sum(smooth16)

    ml_copy.wait()
    mlacc = zeros
    msacc = zeros
    for k in range(_TPW // _L):
        sl = pl.ds(k * _L, _L)
        m16 = mask_v[sl]
        mlacc = mlacc + mlvals_v[sl] * m16
        msacc = msacc + m16
    ml_s = jnp.sum(mlacc)
    msum_s = jnp.sum(msacc)

    stage = jnp.where(iota == 0, ml_s, 0.0)
    stage = stage + jnp.where(iota == 1, smooth_acc, 0.0)
    stage = stage + jnp.where(iota == 2, msum_s, 0.0)
    stage_v[...] = stage
    out_copy = pltpu.async_copy(stage_v, out.at[wid], sem_out)
    out_copy.wait()


@jax.jit
def _sc_partials(inp2, inpflat, tgt, tgt_pad, maskv, sim):
    mesh = plsc.VectorSubcoreMesh(core_axis_name="c", subcore_axis_name="s",
                                  num_cores=_NC, num_subcores=_NS)
    f = pl.kernel(
        _sc_body,
        out_type=jax.ShapeDtypeStruct((_NW, _L), jnp.float32),
        mesh=mesh,
        compiler_params=pltpu.CompilerParams(needs_layout_passes=False,
                                             use_tc_tiling_on_sc=False),
        scratch_types=[
            pltpu.VMEM((_TPW,), jnp.int32),
            pltpu.VMEM((_NCHUNK, 8), jnp.int32),
            pltpu.VMEM((_TPW,), jnp.int32),
            pltpu.VMEM((_TPW,), jnp.float32),
            pltpu.VMEM((_TPW,), jnp.float32),
            pltpu.VMEM((2, _K, _V), jnp.float32),
            pltpu.VMEM((2, _K, _V), jnp.float32),
            pltpu.VMEM((_L,), jnp.float32),
            pltpu.SemaphoreType.DMA,
            pltpu.SemaphoreType.DMA,
            pltpu.SemaphoreType.DMA,
            pltpu.SemaphoreType.DMA,
            pltpu.SemaphoreType.DMA,
            pltpu.SemaphoreType.DMA,
        ],
    )
    return f(inp2, inpflat, tgt, tgt_pad, maskv, sim)


def kernel(input, target, mask, sim_matrix):
    inp2 = input.reshape(_N, _V)
    inpflat = input.reshape(_N * _V)
    tgt = target.reshape(_N).astype(jnp.int32)
    tgt_pad = jnp.pad(tgt.reshape(_N // _K, _K), ((0, 0), (0, 8 - _K)))
    maskv = mask.reshape(_N)
    parts = _sc_partials(inp2, inpflat, tgt, tgt_pad, maskv, sim_matrix)
    ml_sum = jnp.sum(parts[:, 0])
    smooth_sum = jnp.sum(parts[:, 1])
    msum = jnp.sum(parts[:, 2])
    ml_output = -ml_sum / msum
    output = _ALPHA * (-smooth_sum / msum) + (1.0 - _ALPHA) * ml_output
    return (ml_output, output)


# speedup vs baseline: 1.0038x; 1.0038x over previous
import jax
import jax.numpy as jnp
from jax import lax
from jax.experimental import pallas as pl
from jax.experimental.pallas import tpu as pltpu
from jax.experimental.pallas import tpu_sc as plsc

_B, _T, _V = 160, 16, 10000
_TAU = 0.13
_ALPHA = 0.7

_NC, _NS, _L = 2, 16, 16
_NW = _NC * _NS
_N = _B * _T
_TPW = _N // _NW
_K = 2
_NCHUNK = _TPW // _K
_NVEC = _V // _L


def _sc_body(inp2, inpflat, tgt, tgt_pad, maskv, sim, out,
             idx_v, idx2_v, flatidx_v, mask_v, mlvals_v, rows_v, inprows_v,
             stage_v, sem_r0, sem_r1, sem_i0, sem_i1, sem_ml, sem_out):
    wid = lax.axis_index("s") * _NC + lax.axis_index("c")
    base = wid * _TPW
    iota = lax.broadcasted_iota(jnp.int32, (_L,), 0)

    pltpu.sync_copy(tgt.at[pl.ds(base, _TPW)], idx_v)
    pltpu.sync_copy(tgt_pad.at[pl.ds(wid * _NCHUNK, _NCHUNK)], idx2_v)
    pltpu.sync_copy(maskv.at[pl.ds(base, _TPW)], mask_v)

    for k in range(_TPW // _L):
        sl = pl.ds(k * _L, _L)
        t16 = idx_v[sl]
        rowid = iota + (base + k * _L)
        flatidx_v[sl] = rowid * _V + t16
    ml_copy = pltpu.async_copy(inpflat.at[flatidx_v], mlvals_v, sem_ml)

    sem_r = (sem_r0, sem_r1)
    sem_i = (sem_i0, sem_i1)

    handles = {}

    def issue_tracked(g, b):
        h1 = pltpu.async_copy(sim.at[idx2_v.at[g, pl.ds(0, _K)]],
                              rows_v.at[b], sem_r[b])
        h2 = pltpu.async_copy(inp2.at[pl.ds(base + g * _K, _K)],
                              inprows_v.at[b], sem_i[b])
        handles[g] = (h1, h2)

    issue_tracked(0, 0)
    issue_tracked(1, 1)

    zeros = jnp.zeros((_L,), jnp.float32)
    inv_tau = jnp.float32(1.0 / _TAU)
    smooth16 = zeros
    dacc = zeros
    sacc = zeros
    chunks_per_group = _L // _K

    for g in range(_NCHUNK):
        b = g % 2
        h1, h2 = handles.pop(g)
        h1.wait()
        h2.wait()
        @plsc.parallel_loop(0, _V, step=_L, unroll=8,
                            carry=(zeros, zeros, zeros, zeros))
        def carry_out(off, carry):
            d0, s0, d1, s1 = carry
            sl = pl.ds(off, _L)
            e0 = jnp.exp(rows_v[b, 0, sl] * inv_tau)
            e1 = jnp.exp(rows_v[b, 1, sl] * inv_tau)
            return (d0 + e0 * inprows_v[b, 0, sl], s0 + e0,
                    d1 + e1 * inprows_v[b, 1, sl], s1 + e1)

        d0, s0, d1, s1 = carry_out
        for tok, dv, sv in ((0, d0, s0), (1, d1, s1)):
            tokidx = g * _K + tok
            lane = tokidx % _L
            dacc = dacc + jnp.where(iota == lane, jnp.sum(dv), 0.0)
            sacc = sacc + jnp.where(iota == lane, jnp.sum(sv), 0.0)
        if g % chunks_per_group == chunks_per_group - 1:
            grp = g // chunks_per_group
            m16 = mask_v[pl.ds(grp * _L, _L)]
            smooth16 = smooth16 + m16 * dacc / sacc
            dacc = zeros
            sacc = zeros
        if g + 2 < _NCHUNK:
            issue_tracked(g + 2, b)
    smooth_acc = jnp.sum(smooth16)

    ml_copy.wait()
    mlacc = zeros
    msacc = zeros
    for k in range(_TPW // _L):
        sl = pl.ds(k * _L, _L)
        m16 = mask_v[sl]
        mlacc = mlacc + mlvals_v[sl] * m16
        msacc = msacc + m16
    ml_s = jnp.sum(mlacc)
    msum_s = jnp.sum(msacc)

    stage = jnp.where(iota == 0, ml_s, 0.0)
    stage = stage + jnp.where(iota == 1, smooth_acc, 0.0)
    stage = stage + jnp.where(iota == 2, msum_s, 0.0)
    stage_v[...] = stage
    out_copy = pltpu.async_copy(stage_v, out.at[wid], sem_out)
    out_copy.wait()


@jax.jit
def _sc_partials(inp2, inpflat, tgt, tgt_pad, maskv, sim):
    mesh = plsc.VectorSubcoreMesh(core_axis_name="c", subcore_axis_name="s",
                                  num_cores=_NC, num_subcores=_NS)
    f = pl.kernel(
        _sc_body,
        out_type=jax.ShapeDtypeStruct((_NW, _L), jnp.float32),
        mesh=mesh,
        compiler_params=pltpu.CompilerParams(needs_layout_passes=False,
                                             use_tc_tiling_on_sc=False),
        scratch_types=[
            pltpu.VMEM((_TPW,), jnp.int32),
            pltpu.VMEM((_NCHUNK, 8), jnp.int32),
            pltpu.VMEM((_TPW,), jnp.int32),
            pltpu.VMEM((_TPW,), jnp.float32),
            pltpu.VMEM((_TPW,), jnp.float32),
            pltpu.VMEM((2, _K, _V), jnp.float32),
            pltpu.VMEM((2, _K, _V), jnp.float32),
            pltpu.VMEM((_L,), jnp.float32),
            pltpu.SemaphoreType.DMA,
            pltpu.SemaphoreType.DMA,
            pltpu.SemaphoreType.DMA,
            pltpu.SemaphoreType.DMA,
            pltpu.SemaphoreType.DMA,
            pltpu.SemaphoreType.DMA,
        ],
    )
    return f(inp2, inpflat, tgt, tgt_pad, maskv, sim)


def kernel(input, target, mask, sim_matrix):
    inp2 = input.reshape(_N, _V)
    inpflat = input.reshape(_N * _V)
    tgt = target.reshape(_N).astype(jnp.int32)
    tgt_pad = jnp.pad(tgt.reshape(_N // _K, _K), ((0, 0), (0, 8 - _K)))
    maskv = mask.reshape(_N)
    parts = _sc_partials(inp2, inpflat, tgt, tgt_pad, maskv, sim_matrix)
    ml_sum = jnp.sum(parts[:, 0])
    smooth_sum = jnp.sum(parts[:, 1])
    msum = jnp.sum(parts[:, 2])
    ml_output = -ml_sum / msum
    output = _ALPHA * (-smooth_sum / msum) + (1.0 - _ALPHA) * ml_output
    return (ml_output, output)


# speedup vs baseline: 1.0202x; 1.0163x over previous
import jax
import jax.numpy as jnp
from jax import lax
from jax.experimental import pallas as pl
from jax.experimental.pallas import tpu as pltpu
from jax.experimental.pallas import tpu_sc as plsc

_B, _T, _V = 160, 16, 10000
_TAU = 0.13
_ALPHA = 0.7

_NC, _NS, _L = 2, 16, 16
_NW = _NC * _NS
_N = _B * _T
_TPW = _N // _NW
_K = 1
_NBUF = 4
_NCHUNK = _TPW // _K
_NVEC = _V // _L


def _sc_body(inp2, inpflat, tgt, tgt_pad, maskv, sim, out,
             idx_v, idx2_v, flatidx_v, mask_v, mlvals_v, rows_v, inprows_v,
             stage_v, sem_r0, sem_r1, sem_r2, sem_r3,
             sem_i0, sem_i1, sem_i2, sem_i3, sem_ml, sem_out):
    wid = lax.axis_index("s") * _NC + lax.axis_index("c")
    base = wid * _TPW
    iota = lax.broadcasted_iota(jnp.int32, (_L,), 0)

    pltpu.sync_copy(tgt.at[pl.ds(base, _TPW)], idx_v)
    pltpu.sync_copy(tgt_pad.at[pl.ds(wid * _NCHUNK, _NCHUNK)], idx2_v)
    pltpu.sync_copy(maskv.at[pl.ds(base, _TPW)], mask_v)

    for k in range(_TPW // _L):
        sl = pl.ds(k * _L, _L)
        t16 = idx_v[sl]
        rowid = iota + (base + k * _L)
        flatidx_v[sl] = rowid * _V + t16
    ml_copy = pltpu.async_copy(inpflat.at[flatidx_v], mlvals_v, sem_ml)

    sem_r = (sem_r0, sem_r1, sem_r2, sem_r3)
    sem_i = (sem_i0, sem_i1, sem_i2, sem_i3)

    handles = {}

    def issue_tracked(g, b):
        h1 = pltpu.async_copy(sim.at[idx2_v.at[g, pl.ds(0, _K)]],
                              rows_v.at[b], sem_r[b])
        h2 = pltpu.async_copy(inp2.at[pl.ds(base + g * _K, _K)],
                              inprows_v.at[b], sem_i[b])
        handles[g] = (h1, h2)

    for g0 in range(_NBUF):
        issue_tracked(g0, g0)

    zeros = jnp.zeros((_L,), jnp.float32)
    inv_tau = jnp.float32(1.0 / _TAU)
    smooth16 = zeros
    dacc = zeros
    sacc = zeros
    chunks_per_group = _L // _K

    for g in range(_NCHUNK):
        b = g % _NBUF
        h1, h2 = handles.pop(g)
        h1.wait()
        h2.wait()
        @plsc.parallel_loop(0, _V, step=_L, unroll=8, carry=(zeros, zeros))
        def carry_out(off, carry):
            d0, s0 = carry
            sl = pl.ds(off, _L)
            e0 = jnp.exp(rows_v[b, 0, sl] * inv_tau)
            return (d0 + e0 * inprows_v[b, 0, sl], s0 + e0)

        dv, sv = carry_out
        lane = g % _L
        dacc = dacc + jnp.where(iota == lane, jnp.sum(dv), 0.0)
        sacc = sacc + jnp.where(iota == lane, jnp.sum(sv), 0.0)
        if g % chunks_per_group == chunks_per_group - 1:
            grp = g // chunks_per_group
            m16 = mask_v[pl.ds(grp * _L, _L)]
            smooth16 = smooth16 + m16 * dacc / sacc
            dacc = zeros
            sacc = zeros
        if g + _NBUF < _NCHUNK:
            issue_tracked(g + _NBUF, b)
    smooth_acc = jnp.sum(smooth16)

    ml_copy.wait()
    mlacc = zeros
    msacc = zeros
    for k in range(_TPW // _L):
        sl = pl.ds(k * _L, _L)
        m16 = mask_v[sl]
        mlacc = mlacc + mlvals_v[sl] * m16
        msacc = msacc + m16
    ml_s = jnp.sum(mlacc)
    msum_s = jnp.sum(msacc)

    stage = jnp.where(iota == 0, ml_s, 0.0)
    stage = stage + jnp.where(iota == 1, smooth_acc, 0.0)
    stage = stage + jnp.where(iota == 2, msum_s, 0.0)
    stage_v[...] = stage
    out_copy = pltpu.async_copy(stage_v, out.at[wid], sem_out)
    out_copy.wait()


@jax.jit
def _sc_partials(inp2, inpflat, tgt, tgt_pad, maskv, sim):
    mesh = plsc.VectorSubcoreMesh(core_axis_name="c", subcore_axis_name="s",
                                  num_cores=_NC, num_subcores=_NS)
    f = pl.kernel(
        _sc_body,
        out_type=jax.ShapeDtypeStruct((_NW, _L), jnp.float32),
        mesh=mesh,
        compiler_params=pltpu.CompilerParams(needs_layout_passes=False,
                                             use_tc_tiling_on_sc=False),
        scratch_types=[
            pltpu.VMEM((_TPW,), jnp.int32),
            pltpu.VMEM((_NCHUNK, 8), jnp.int32),
            pltpu.VMEM((_TPW,), jnp.int32),
            pltpu.VMEM((_TPW,), jnp.float32),
            pltpu.VMEM((_TPW,), jnp.float32),
            pltpu.VMEM((_NBUF, _K, _V), jnp.float32),
            pltpu.VMEM((_NBUF, _K, _V), jnp.float32),
            pltpu.VMEM((_L,), jnp.float32),
            pltpu.SemaphoreType.DMA,
            pltpu.SemaphoreType.DMA,
            pltpu.SemaphoreType.DMA,
            pltpu.SemaphoreType.DMA,
            pltpu.SemaphoreType.DMA,
            pltpu.SemaphoreType.DMA,
            pltpu.SemaphoreType.DMA,
            pltpu.SemaphoreType.DMA,
            pltpu.SemaphoreType.DMA,
            pltpu.SemaphoreType.DMA,
        ],
    )
    return f(inp2, inpflat, tgt, tgt_pad, maskv, sim)


def kernel(input, target, mask, sim_matrix):
    inp2 = input.reshape(_N, _V)
    inpflat = input.reshape(_N * _V)
    tgt = target.reshape(_N).astype(jnp.int32)
    tgt_pad = jnp.pad(tgt.reshape(_N // _K, _K), ((0, 0), (0, 8 - _K)))
    maskv = mask.reshape(_N)
    parts = _sc_partials(inp2, inpflat, tgt, tgt_pad, maskv, sim_matrix)
    ml_sum = jnp.sum(parts[:, 0])
    smooth_sum = jnp.sum(parts[:, 1])
    msum = jnp.sum(parts[:, 2])
    ml_output = -ml_sum / msum
    output = _ALPHA * (-smooth_sum / msum) + (1.0 - _ALPHA) * ml_output
    return (ml_output, output)


# speedup vs baseline: 1.7812x; 1.7459x over previous
import jax
import jax.numpy as jnp
from jax import lax
from jax.experimental import pallas as pl
from jax.experimental.pallas import tpu as pltpu
from jax.experimental.pallas import tpu_sc as plsc

_B, _T, _V = 160, 16, 10000
_TAU = 0.13
_ALPHA = 0.7

_NC, _NS, _L = 2, 16, 16
_NW = _NC * _NS
_N = _B * _T
_TPW = _N // _NW

_R = 8
_NSTEP = _N // _R


def _sc_nll_body(inpflat, tgt, maskv, out,
                 idx_v, flatidx_v, mask_v, mlvals_v, stage_v, sem_ml, sem_out):
    wid = lax.axis_index("s") * _NC + lax.axis_index("c")
    base = wid * _TPW
    iota = lax.broadcasted_iota(jnp.int32, (_L,), 0)

    pltpu.sync_copy(tgt.at[pl.ds(base, _TPW)], idx_v)
    pltpu.sync_copy(maskv.at[pl.ds(base, _TPW)], mask_v)

    for k in range(_TPW // _L):
        sl = pl.ds(k * _L, _L)
        rowid = iota + (base + k * _L)
        flatidx_v[sl] = rowid * _V + idx_v[sl]
    pltpu.async_copy(inpflat.at[flatidx_v], mlvals_v, sem_ml).wait()

    zeros = jnp.zeros((_L,), jnp.float32)
    mlacc = zeros
    msacc = zeros
    for k in range(_TPW // _L):
        sl = pl.ds(k * _L, _L)
        m16 = mask_v[sl]
        mlacc = mlacc + mlvals_v[sl] * m16
        msacc = msacc + m16
    stage = jnp.where(iota == 0, jnp.sum(mlacc), 0.0)
    stage = stage + jnp.where(iota == 1, jnp.sum(msacc), 0.0)
    stage_v[...] = stage
    pltpu.async_copy(stage_v, out.at[wid], sem_out).wait()


@jax.jit
def _sc_nll(inpflat, tgt, maskv):
    mesh = plsc.VectorSubcoreMesh(core_axis_name="c", subcore_axis_name="s",
                                  num_cores=_NC, num_subcores=_NS)
    f = pl.kernel(
        _sc_nll_body,
        out_type=jax.ShapeDtypeStruct((_NW, _L), jnp.float32),
        mesh=mesh,
        compiler_params=pltpu.CompilerParams(needs_layout_passes=False,
                                             use_tc_tiling_on_sc=False),
        scratch_types=[
            pltpu.VMEM((_TPW,), jnp.int32),
            pltpu.VMEM((_TPW,), jnp.int32),
            pltpu.VMEM((_TPW,), jnp.float32),
            pltpu.VMEM((_TPW,), jnp.float32),
            pltpu.VMEM((_L,), jnp.float32),
            pltpu.SemaphoreType.DMA,
            pltpu.SemaphoreType.DMA,
        ],
    )
    return f(inpflat, tgt, maskv)


def _tc_body(t_ref, inp_ref, mask_ref, sim_ref, o_sm_ref, buf, sems):
    i = pl.program_id(0)
    inv_tau = jnp.float32(1.0 / _TAU)

    def issue(step, slot):
        for r in range(_R):
            t = t_ref[step * _R + r]
            pltpu.make_async_copy(sim_ref.at[t], buf.at[slot, r],
                                  sems.at[slot, r]).start()

    @pl.when(i == 0)
    def _():
        issue(0, 0)

    @pl.when(i + 1 < _NSTEP)
    def _():
        issue(i + 1, (i + 1) % 2)

    slot = i % 2
    for r in range(_R):
        t = t_ref[i * _R + r]
        pltpu.make_async_copy(sim_ref.at[t], buf.at[slot, r],
                              sems.at[slot, r]).wait()

    x = buf[slot]
    e = jnp.exp(x * inv_tau)
    inp8 = inp_ref[...]
    m8 = mask_ref[...]
    s8 = jnp.sum(e, axis=1, keepdims=True)
    d8 = jnp.sum(e * inp8, axis=1, keepdims=True)
    contrib = jnp.sum(m8 * d8 / s8)

    @pl.when(i == 0)
    def _():
        o_sm_ref[0, 0] = 0.0

    o_sm_ref[0, 0] += contrib


@jax.jit
def _tc_smooth(tgt, inp2, mask2d, sim):
    return pl.pallas_call(
        _tc_body,
        grid_spec=pltpu.PrefetchScalarGridSpec(
            num_scalar_prefetch=1,
            grid=(_NSTEP,),
            in_specs=[
                pl.BlockSpec((_R, _V), lambda i, t: (i, 0)),
                pl.BlockSpec((_R, 1), lambda i, t: (i, 0)),
                pl.BlockSpec(memory_space=pltpu.MemorySpace.HBM),
            ],
            out_specs=pl.BlockSpec((1, 1), lambda i, t: (0, 0),
                                   memory_space=pltpu.MemorySpace.SMEM),
            scratch_shapes=[
                pltpu.VMEM((2, _R, _V), jnp.float32),
                pltpu.SemaphoreType.DMA((2, _R)),
            ],
        ),
        out_shape=jax.ShapeDtypeStruct((1, 1), jnp.float32),
        compiler_params=pltpu.CompilerParams(
            dimension_semantics=("arbitrary",)),
    )(tgt, inp2, mask2d, sim)


def kernel(input, target, mask, sim_matrix):
    inp2 = input.reshape(_N, _V)
    inpflat = input.reshape(_N * _V)
    tgt = target.reshape(_N).astype(jnp.int32)
    maskv = mask.reshape(_N)
    mask2d = mask.reshape(_N, 1)

    parts = _sc_nll(inpflat, tgt, maskv)
    smooth_sum = _tc_smooth(tgt, inp2, mask2d, sim_matrix)[0, 0]

    ml_sum = jnp.sum(parts[:, 0])
    msum = jnp.sum(parts[:, 1])
    ml_output = -ml_sum / msum
    output = _ALPHA * (-smooth_sum / msum) + (1.0 - _ALPHA) * ml_output
    return (ml_output, output)


# speedup vs baseline: 1.9656x; 1.1035x over previous
import jax
import jax.numpy as jnp
from jax import lax
from jax.experimental import pallas as pl
from jax.experimental.pallas import tpu as pltpu
from jax.experimental.pallas import tpu_sc as plsc

_B, _T, _V = 160, 16, 10000
_TAU = 0.13
_ALPHA = 0.7

_NC, _NS, _L = 2, 16, 16
_NW = _NC * _NS
_N = _B * _T
_TPW = _N // _NW

_R = 8
_NSTEP = _N // _R
_NBUF = 4


def _sc_nll_body(inpflat, tgt, maskv, out,
                 idx_v, flatidx_v, mask_v, mlvals_v, stage_v, sem_ml, sem_out):
    wid = lax.axis_index("s") * _NC + lax.axis_index("c")
    base = wid * _TPW
    iota = lax.broadcasted_iota(jnp.int32, (_L,), 0)

    pltpu.sync_copy(tgt.at[pl.ds(base, _TPW)], idx_v)
    pltpu.sync_copy(maskv.at[pl.ds(base, _TPW)], mask_v)

    for k in range(_TPW // _L):
        sl = pl.ds(k * _L, _L)
        rowid = iota + (base + k * _L)
        flatidx_v[sl] = rowid * _V + idx_v[sl]
    pltpu.async_copy(inpflat.at[flatidx_v], mlvals_v, sem_ml).wait()

    zeros = jnp.zeros((_L,), jnp.float32)
    mlacc = zeros
    msacc = zeros
    for k in range(_TPW // _L):
        sl = pl.ds(k * _L, _L)
        m16 = mask_v[sl]
        mlacc = mlacc + mlvals_v[sl] * m16
        msacc = msacc + m16
    stage = jnp.where(iota == 0, jnp.sum(mlacc), 0.0)
    stage = stage + jnp.where(iota == 1, jnp.sum(msacc), 0.0)
    stage_v[...] = stage
    pltpu.async_copy(stage_v, out.at[wid], sem_out).wait()


@jax.jit
def _sc_nll(inpflat, tgt, maskv):
    mesh = plsc.VectorSubcoreMesh(core_axis_name="c", subcore_axis_name="s",
                                  num_cores=_NC, num_subcores=_NS)
    f = pl.kernel(
        _sc_nll_body,
        out_type=jax.ShapeDtypeStruct((_NW, _L), jnp.float32),
        mesh=mesh,
        compiler_params=pltpu.CompilerParams(needs_layout_passes=False,
                                             use_tc_tiling_on_sc=False),
        scratch_types=[
            pltpu.VMEM((_TPW,), jnp.int32),
            pltpu.VMEM((_TPW,), jnp.int32),
            pltpu.VMEM((_TPW,), jnp.float32),
            pltpu.VMEM((_TPW,), jnp.float32),
            pltpu.VMEM((_L,), jnp.float32),
            pltpu.SemaphoreType.DMA,
            pltpu.SemaphoreType.DMA,
        ],
    )
    return f(inpflat, tgt, maskv)


def _tc_body(t_ref, inp_ref, mask_ref, sim_ref, o_sm_ref, buf, sems):
    i = pl.program_id(0)
    inv_tau = jnp.float32(1.0 / _TAU)

    def issue(step, slot):
        for r in range(_R):
            t = t_ref[step * _R + r]
            pltpu.make_async_copy(sim_ref.at[t], buf.at[slot, r],
                                  sems.at[slot, r]).start()

    @pl.when(i == 0)
    def _():
        for s in range(_NBUF - 1):
            issue(s, s)

    @pl.when(i + _NBUF - 1 < _NSTEP)
    def _():
        issue(i + _NBUF - 1, lax.rem(i + _NBUF - 1, _NBUF))

    slot = lax.rem(i, _NBUF)
    for r in range(_R):
        t = t_ref[i * _R + r]
        pltpu.make_async_copy(sim_ref.at[t], buf.at[slot, r],
                              sems.at[slot, r]).wait()

    x = buf[slot]
    e = jnp.exp(x * inv_tau)
    inp8 = inp_ref[...]
    m8 = mask_ref[...]
    s8 = jnp.sum(e, axis=1, keepdims=True)
    d8 = jnp.sum(e * inp8, axis=1, keepdims=True)
    contrib = jnp.sum(m8 * d8 / s8)

    @pl.when(i == 0)
    def _():
        o_sm_ref[0, 0] = 0.0

    o_sm_ref[0, 0] += contrib


@jax.jit
def _tc_smooth(tgt, inp2, mask2d, sim):
    return pl.pallas_call(
        _tc_body,
        grid_spec=pltpu.PrefetchScalarGridSpec(
            num_scalar_prefetch=1,
            grid=(_NSTEP,),
            in_specs=[
                pl.BlockSpec((_R, _V), lambda i, t: (i, 0)),
                pl.BlockSpec((_R, 1), lambda i, t: (i, 0)),
                pl.BlockSpec(memory_space=pltpu.MemorySpace.HBM),
            ],
            out_specs=pl.BlockSpec((1, 1), lambda i, t: (0, 0),
                                   memory_space=pltpu.MemorySpace.SMEM),
            scratch_shapes=[
                pltpu.VMEM((_NBUF, _R, _V), jnp.float32),
                pltpu.SemaphoreType.DMA((_NBUF, _R)),
            ],
        ),
        out_shape=jax.ShapeDtypeStruct((1, 1), jnp.float32),
        compiler_params=pltpu.CompilerParams(
            dimension_semantics=("arbitrary",)),
    )(tgt, inp2, mask2d, sim)


def kernel(input, target, mask, sim_matrix):
    inp2 = input.reshape(_N, _V)
    inpflat = input.reshape(_N * _V)
    tgt = target.reshape(_N).astype(jnp.int32)
    maskv = mask.reshape(_N)
    mask2d = mask.reshape(_N, 1)

    parts = _sc_nll(inpflat, tgt, maskv)
    smooth_sum = _tc_smooth(tgt, inp2, mask2d, sim_matrix)[0, 0]

    ml_sum = jnp.sum(parts[:, 0])
    msum = jnp.sum(parts[:, 1])
    ml_output = -ml_sum / msum
    output = _ALPHA * (-smooth_sum / msum) + (1.0 - _ALPHA) * ml_output
    return (ml_output, output)


# speedup vs baseline: 2.6850x; 1.3660x over previous
import jax
import jax.numpy as jnp
from jax import lax
from jax.experimental import pallas as pl
from jax.experimental.pallas import tpu as pltpu
from jax.experimental.pallas import tpu_sc as plsc

_B, _T, _V = 160, 16, 10000
_TAU = 0.13
_ALPHA = 0.7

_NC, _NS, _L = 2, 16, 16
_NW = _NC * _NS
_N = _B * _T
_TPW = _N // _NW

_R = 16
_NSTEP = _N // _R
_NBUF = 4


def _sc_nll_body(inpflat, tgt, maskv, out,
                 idx_v, flatidx_v, mask_v, mlvals_v, stage_v, sem_ml, sem_out):
    wid = lax.axis_index("s") * _NC + lax.axis_index("c")
    base = wid * _TPW
    iota = lax.broadcasted_iota(jnp.int32, (_L,), 0)

    pltpu.sync_copy(tgt.at[pl.ds(base, _TPW)], idx_v)
    pltpu.sync_copy(maskv.at[pl.ds(base, _TPW)], mask_v)

    for k in range(_TPW // _L):
        sl = pl.ds(k * _L, _L)
        rowid = iota + (base + k * _L)
        flatidx_v[sl] = rowid * _V + idx_v[sl]
    pltpu.async_copy(inpflat.at[flatidx_v], mlvals_v, sem_ml).wait()

    zeros = jnp.zeros((_L,), jnp.float32)
    mlacc = zeros
    msacc = zeros
    for k in range(_TPW // _L):
        sl = pl.ds(k * _L, _L)
        m16 = mask_v[sl]
        mlacc = mlacc + mlvals_v[sl] * m16
        msacc = msacc + m16
    stage = jnp.where(iota == 0, jnp.sum(mlacc), 0.0)
    stage = stage + jnp.where(iota == 1, jnp.sum(msacc), 0.0)
    stage_v[...] = stage
    pltpu.async_copy(stage_v, out.at[wid], sem_out).wait()


@jax.jit
def _sc_nll(inpflat, tgt, maskv):
    mesh = plsc.VectorSubcoreMesh(core_axis_name="c", subcore_axis_name="s",
                                  num_cores=_NC, num_subcores=_NS)
    f = pl.kernel(
        _sc_nll_body,
        out_type=jax.ShapeDtypeStruct((_NW, _L), jnp.float32),
        mesh=mesh,
        compiler_params=pltpu.CompilerParams(needs_layout_passes=False,
                                             use_tc_tiling_on_sc=False),
        scratch_types=[
            pltpu.VMEM((_TPW,), jnp.int32),
            pltpu.VMEM((_TPW,), jnp.int32),
            pltpu.VMEM((_TPW,), jnp.float32),
            pltpu.VMEM((_TPW,), jnp.float32),
            pltpu.VMEM((_L,), jnp.float32),
            pltpu.SemaphoreType.DMA,
            pltpu.SemaphoreType.DMA,
        ],
    )
    return f(inpflat, tgt, maskv)


def _tc_body(t_ref, inp_ref, mask_ref, sim_ref, o_sm_ref, buf, sems):
    i = pl.program_id(0)
    inv_tau = jnp.float32(1.0 / _TAU)

    def issue(step, slot):
        for r in range(_R):
            t = t_ref[step * _R + r]
            pltpu.make_async_copy(sim_ref.at[t], buf.at[slot, r],
                                  sems.at[slot]).start()

    @pl.when(i == 0)
    def _():
        for s in range(_NBUF - 1):
            issue(s, s)

    @pl.when(i + _NBUF - 1 < _NSTEP)
    def _():
        issue(i + _NBUF - 1, lax.rem(i + _NBUF - 1, _NBUF))

    slot = lax.rem(i, _NBUF)
    pltpu.make_async_copy(sim_ref.at[pl.ds(0, _R)], buf.at[slot],
                          sems.at[slot]).wait()

    x = buf[slot]
    e = jnp.exp(x * inv_tau)
    inp8 = inp_ref[...]
    m8 = mask_ref[...]
    s8 = jnp.sum(e, axis=1, keepdims=True)
    d8 = jnp.sum(e * inp8, axis=1, keepdims=True)
    contrib = jnp.sum(m8 * d8 / s8)

    @pl.when(i == 0)
    def _():
        o_sm_ref[0, 0] = 0.0

    o_sm_ref[0, 0] += contrib


@jax.jit
def _tc_smooth(tgt, inp2, mask2d, sim):
    return pl.pallas_call(
        _tc_body,
        grid_spec=pltpu.PrefetchScalarGridSpec(
            num_scalar_prefetch=1,
            grid=(_NSTEP,),
            in_specs=[
                pl.BlockSpec((_R, _V), lambda i, t: (i, 0)),
                pl.BlockSpec((_R, 1), lambda i, t: (i, 0)),
                pl.BlockSpec(memory_space=pltpu.MemorySpace.HBM),
            ],
            out_specs=pl.BlockSpec((1, 1), lambda i, t: (0, 0),
                                   memory_space=pltpu.MemorySpace.SMEM),
            scratch_shapes=[
                pltpu.VMEM((_NBUF, _R, _V), jnp.float32),
                pltpu.SemaphoreType.DMA((_NBUF,)),
            ],
        ),
        out_shape=jax.ShapeDtypeStruct((1, 1), jnp.float32),
        compiler_params=pltpu.CompilerParams(
            dimension_semantics=("arbitrary",)),
    )(tgt, inp2, mask2d, sim)


def kernel(input, target, mask, sim_matrix):
    inp2 = input.reshape(_N, _V)
    inpflat = input.reshape(_N * _V)
    tgt = target.reshape(_N).astype(jnp.int32)
    maskv = mask.reshape(_N)
    mask2d = mask.reshape(_N, 1)

    parts = _sc_nll(inpflat, tgt, maskv)
    smooth_sum = _tc_smooth(tgt, inp2, mask2d, sim_matrix)[0, 0]

    ml_sum = jnp.sum(parts[:, 0])
    msum = jnp.sum(parts[:, 1])
    ml_output = -ml_sum / msum
    output = _ALPHA * (-smooth_sum / msum) + (1.0 - _ALPHA) * ml_output
    return (ml_output, output)


# speedup vs baseline: 3.2765x; 1.2203x over previous
import jax
import jax.numpy as jnp
from jax import lax
from jax.experimental import pallas as pl
from jax.experimental.pallas import tpu as pltpu
from jax.experimental.pallas import tpu_sc as plsc

_B, _T, _V = 160, 16, 10000
_TAU = 0.13
_ALPHA = 0.7

_NC, _NS, _L = 2, 16, 16
_NW = _NC * _NS
_N = _B * _T
_TPW = _N // _NW

_R = 32
_NSTEP = _N // _R
_NBUF = 4


def _sc_nll_body(inpflat, tgt, maskv, out,
                 idx_v, flatidx_v, mask_v, mlvals_v, stage_v, sem_ml, sem_out):
    wid = lax.axis_index("s") * _NC + lax.axis_index("c")
    base = wid * _TPW
    iota = lax.broadcasted_iota(jnp.int32, (_L,), 0)

    pltpu.sync_copy(tgt.at[pl.ds(base, _TPW)], idx_v)
    pltpu.sync_copy(maskv.at[pl.ds(base, _TPW)], mask_v)

    for k in range(_TPW // _L):
        sl = pl.ds(k * _L, _L)
        rowid = iota + (base + k * _L)
        flatidx_v[sl] = rowid * _V + idx_v[sl]
    pltpu.async_copy(inpflat.at[flatidx_v], mlvals_v, sem_ml).wait()

    zeros = jnp.zeros((_L,), jnp.float32)
    mlacc = zeros
    msacc = zeros
    for k in range(_TPW // _L):
        sl = pl.ds(k * _L, _L)
        m16 = mask_v[sl]
        mlacc = mlacc + mlvals_v[sl] * m16
        msacc = msacc + m16
    stage = jnp.where(iota == 0, jnp.sum(mlacc), 0.0)
    stage = stage + jnp.where(iota == 1, jnp.sum(msacc), 0.0)
    stage_v[...] = stage
    pltpu.async_copy(stage_v, out.at[wid], sem_out).wait()


@jax.jit
def _sc_nll(inpflat, tgt, maskv):
    mesh = plsc.VectorSubcoreMesh(core_axis_name="c", subcore_axis_name="s",
                                  num_cores=_NC, num_subcores=_NS)
    f = pl.kernel(
        _sc_nll_body,
        out_type=jax.ShapeDtypeStruct((_NW, _L), jnp.float32),
        mesh=mesh,
        compiler_params=pltpu.CompilerParams(needs_layout_passes=False,
                                             use_tc_tiling_on_sc=False),
        scratch_types=[
            pltpu.VMEM((_TPW,), jnp.int32),
            pltpu.VMEM((_TPW,), jnp.int32),
            pltpu.VMEM((_TPW,), jnp.float32),
            pltpu.VMEM((_TPW,), jnp.float32),
            pltpu.VMEM((_L,), jnp.float32),
            pltpu.SemaphoreType.DMA,
            pltpu.SemaphoreType.DMA,
        ],
    )
    return f(inpflat, tgt, maskv)


def _tc_body(t_ref, inp_ref, mask_ref, sim_ref, o_sm_ref, buf, sems):
    i = pl.program_id(0)
    inv_tau = jnp.float32(1.0 / _TAU)

    def issue(step, slot):
        for r in range(_R):
            t = t_ref[step * _R + r]
            pltpu.make_async_copy(sim_ref.at[t], buf.at[slot, r],
                                  sems.at[slot]).start()

    @pl.when(i == 0)
    def _():
        for s in range(_NBUF - 1):
            issue(s, s)

    @pl.when(i + _NBUF - 1 < _NSTEP)
    def _():
        issue(i + _NBUF - 1, lax.rem(i + _NBUF - 1, _NBUF))

    slot = lax.rem(i, _NBUF)
    pltpu.make_async_copy(sim_ref.at[pl.ds(0, _R)], buf.at[slot],
                          sems.at[slot]).wait()

    x = buf[slot]
    e = jnp.exp(x * inv_tau)
    inp8 = inp_ref[...]
    m8 = mask_ref[...]
    s8 = jnp.sum(e, axis=1, keepdims=True)
    d8 = jnp.sum(e * inp8, axis=1, keepdims=True)
    contrib = jnp.sum(m8 * d8 / s8)

    @pl.when(i == 0)
    def _():
        o_sm_ref[0, 0] = 0.0

    o_sm_ref[0, 0] += contrib


@jax.jit
def _tc_smooth(tgt, inp2, mask2d, sim):
    return pl.pallas_call(
        _tc_body,
        grid_spec=pltpu.PrefetchScalarGridSpec(
            num_scalar_prefetch=1,
            grid=(_NSTEP,),
            in_specs=[
                pl.BlockSpec((_R, _V), lambda i, t: (i, 0)),
                pl.BlockSpec((_R, 1), lambda i, t: (i, 0)),
                pl.BlockSpec(memory_space=pltpu.MemorySpace.HBM),
            ],
            out_specs=pl.BlockSpec((1, 1), lambda i, t: (0, 0),
                                   memory_space=pltpu.MemorySpace.SMEM),
            scratch_shapes=[
                pltpu.VMEM((_NBUF, _R, _V), jnp.float32),
                pltpu.SemaphoreType.DMA((_NBUF,)),
            ],
        ),
        out_shape=jax.ShapeDtypeStruct((1, 1), jnp.float32),
        compiler_params=pltpu.CompilerParams(
            dimension_semantics=("arbitrary",)),
    )(tgt, inp2, mask2d, sim)


def kernel(input, target, mask, sim_matrix):
    inp2 = input.reshape(_N, _V)
    inpflat = input.reshape(_N * _V)
    tgt = target.reshape(_N).astype(jnp.int32)
    maskv = mask.reshape(_N)
    mask2d = mask.reshape(_N, 1)

    parts = _sc_nll(inpflat, tgt, maskv)
    smooth_sum = _tc_smooth(tgt, inp2, mask2d, sim_matrix)[0, 0]

    ml_sum = jnp.sum(parts[:, 0])
    msum = jnp.sum(parts[:, 1])
    ml_output = -ml_sum / msum
    output = _ALPHA * (-smooth_sum / msum) + (1.0 - _ALPHA) * ml_output
    return (ml_output, output)


# speedup vs baseline: 3.6103x; 1.1019x over previous
import jax
import jax.numpy as jnp
from jax import lax
from jax.experimental import pallas as pl
from jax.experimental.pallas import tpu as pltpu
from jax.experimental.pallas import tpu_sc as plsc

_B, _T, _V = 160, 16, 10000
_TAU = 0.13
_ALPHA = 0.7

_NC, _NS, _L = 2, 16, 16
_NW = _NC * _NS
_N = _B * _T
_TPW = _N // _NW

_R = 64
_NSTEP = _N // _R
_NBUF = 4


def _sc_nll_body(inpflat, tgt, maskv, out,
                 idx_v, flatidx_v, mask_v, mlvals_v, stage_v, sem_ml, sem_out):
    wid = lax.axis_index("s") * _NC + lax.axis_index("c")
    base = wid * _TPW
    iota = lax.broadcasted_iota(jnp.int32, (_L,), 0)

    pltpu.sync_copy(tgt.at[pl.ds(base, _TPW)], idx_v)
    pltpu.sync_copy(maskv.at[pl.ds(base, _TPW)], mask_v)

    for k in range(_TPW // _L):
        sl = pl.ds(k * _L, _L)
        rowid = iota + (base + k * _L)
        flatidx_v[sl] = rowid * _V + idx_v[sl]
    pltpu.async_copy(inpflat.at[flatidx_v], mlvals_v, sem_ml).wait()

    zeros = jnp.zeros((_L,), jnp.float32)
    mlacc = zeros
    msacc = zeros
    for k in range(_TPW // _L):
        sl = pl.ds(k * _L, _L)
        m16 = mask_v[sl]
        mlacc = mlacc + mlvals_v[sl] * m16
        msacc = msacc + m16
    stage = jnp.where(iota == 0, jnp.sum(mlacc), 0.0)
    stage = stage + jnp.where(iota == 1, jnp.sum(msacc), 0.0)
    stage_v[...] = stage
    pltpu.async_copy(stage_v, out.at[wid], sem_out).wait()


@jax.jit
def _sc_nll(inpflat, tgt, maskv):
    mesh = plsc.VectorSubcoreMesh(core_axis_name="c", subcore_axis_name="s",
                                  num_cores=_NC, num_subcores=_NS)
    f = pl.kernel(
        _sc_nll_body,
        out_type=jax.ShapeDtypeStruct((_NW, _L), jnp.float32),
        mesh=mesh,
        compiler_params=pltpu.CompilerParams(needs_layout_passes=False,
                                             use_tc_tiling_on_sc=False),
        scratch_types=[
            pltpu.VMEM((_TPW,), jnp.int32),
            pltpu.VMEM((_TPW,), jnp.int32),
            pltpu.VMEM((_TPW,), jnp.float32),
            pltpu.VMEM((_TPW,), jnp.float32),
            pltpu.VMEM((_L,), jnp.float32),
            pltpu.SemaphoreType.DMA,
            pltpu.SemaphoreType.DMA,
        ],
    )
    return f(inpflat, tgt, maskv)


def _tc_body(t_ref, inp_ref, mask_ref, sim_ref, o_sm_ref, buf, sems):
    i = pl.program_id(0)
    inv_tau = jnp.float32(1.0 / _TAU)

    def issue(step, slot):
        for r in range(_R):
            t = t_ref[step * _R + r]
            pltpu.make_async_copy(sim_ref.at[t], buf.at[slot, r],
                                  sems.at[slot]).start()

    @pl.when(i == 0)
    def _():
        for s in range(_NBUF - 1):
            issue(s, s)

    @pl.when(i + _NBUF - 1 < _NSTEP)
    def _():
        issue(i + _NBUF - 1, lax.rem(i + _NBUF - 1, _NBUF))

    slot = lax.rem(i, _NBUF)
    pltpu.make_async_copy(sim_ref.at[pl.ds(0, _R)], buf.at[slot],
                          sems.at[slot]).wait()

    x = buf[slot]
    e = jnp.exp(x * inv_tau)
    inp8 = inp_ref[...]
    m8 = mask_ref[...]
    s8 = jnp.sum(e, axis=1, keepdims=True)
    d8 = jnp.sum(e * inp8, axis=1, keepdims=True)
    contrib = jnp.sum(m8 * d8 / s8)

    @pl.when(i == 0)
    def _():
        o_sm_ref[0, 0] = 0.0

    o_sm_ref[0, 0] += contrib


@jax.jit
def _tc_smooth(tgt, inp2, mask2d, sim):
    return pl.pallas_call(
        _tc_body,
        grid_spec=pltpu.PrefetchScalarGridSpec(
            num_scalar_prefetch=1,
            grid=(_NSTEP,),
            in_specs=[
                pl.BlockSpec((_R, _V), lambda i, t: (i, 0)),
                pl.BlockSpec((_R, 1), lambda i, t: (i, 0)),
                pl.BlockSpec(memory_space=pltpu.MemorySpace.HBM),
            ],
            out_specs=pl.BlockSpec((1, 1), lambda i, t: (0, 0),
                                   memory_space=pltpu.MemorySpace.SMEM),
            scratch_shapes=[
                pltpu.VMEM((_NBUF, _R, _V), jnp.float32),
                pltpu.SemaphoreType.DMA((_NBUF,)),
            ],
        ),
        out_shape=jax.ShapeDtypeStruct((1, 1), jnp.float32),
        compiler_params=pltpu.CompilerParams(
            dimension_semantics=("arbitrary",)),
    )(tgt, inp2, mask2d, sim)


def kernel(input, target, mask, sim_matrix):
    inp2 = input.reshape(_N, _V)
    inpflat = input.reshape(_N * _V)
    tgt = target.reshape(_N).astype(jnp.int32)
    maskv = mask.reshape(_N)
    mask2d = mask.reshape(_N, 1)

    parts = _sc_nll(inpflat, tgt, maskv)
    smooth_sum = _tc_smooth(tgt, inp2, mask2d, sim_matrix)[0, 0]

    ml_sum = jnp.sum(parts[:, 0])
    msum = jnp.sum(parts[:, 1])
    ml_output = -ml_sum / msum
    output = _ALPHA * (-smooth_sum / msum) + (1.0 - _ALPHA) * ml_output
    return (ml_output, output)


# speedup vs baseline: 3.6237x; 1.0037x over previous
import jax
import jax.numpy as jnp
from jax import lax
from jax.experimental import pallas as pl
from jax.experimental.pallas import tpu as pltpu
from jax.experimental.pallas import tpu_sc as plsc

_B, _T, _V = 160, 16, 10000
_TAU = 0.13
_ALPHA = 0.7

_NC, _NS, _L = 2, 16, 16
_NW = _NC * _NS
_N = _B * _T
_TPW = _N // _NW

_R = 128
_NSTEP = _N // _R
_NBUF = 4


def _sc_nll_body(inpflat, tgt, maskv, out,
                 idx_v, flatidx_v, mask_v, mlvals_v, stage_v, sem_ml, sem_out):
    wid = lax.axis_index("s") * _NC + lax.axis_index("c")
    base = wid * _TPW
    iota = lax.broadcasted_iota(jnp.int32, (_L,), 0)

    pltpu.sync_copy(tgt.at[pl.ds(base, _TPW)], idx_v)
    pltpu.sync_copy(maskv.at[pl.ds(base, _TPW)], mask_v)

    for k in range(_TPW // _L):
        sl = pl.ds(k * _L, _L)
        rowid = iota + (base + k * _L)
        flatidx_v[sl] = rowid * _V + idx_v[sl]
    pltpu.async_copy(inpflat.at[flatidx_v], mlvals_v, sem_ml).wait()

    zeros = jnp.zeros((_L,), jnp.float32)
    mlacc = zeros
    msacc = zeros
    for k in range(_TPW // _L):
        sl = pl.ds(k * _L, _L)
        m16 = mask_v[sl]
        mlacc = mlacc + mlvals_v[sl] * m16
        msacc = msacc + m16
    stage = jnp.where(iota == 0, jnp.sum(mlacc), 0.0)
    stage = stage + jnp.where(iota == 1, jnp.sum(msacc), 0.0)
    stage_v[...] = stage
    pltpu.async_copy(stage_v, out.at[wid], sem_out).wait()


@jax.jit
def _sc_nll(inpflat, tgt, maskv):
    mesh = plsc.VectorSubcoreMesh(core_axis_name="c", subcore_axis_name="s",
                                  num_cores=_NC, num_subcores=_NS)
    f = pl.kernel(
        _sc_nll_body,
        out_type=jax.ShapeDtypeStruct((_NW, _L), jnp.float32),
        mesh=mesh,
        compiler_params=pltpu.CompilerParams(needs_layout_passes=False,
                                             use_tc_tiling_on_sc=False),
        scratch_types=[
            pltpu.VMEM((_TPW,), jnp.int32),
            pltpu.VMEM((_TPW,), jnp.int32),
            pltpu.VMEM((_TPW,), jnp.float32),
            pltpu.VMEM((_TPW,), jnp.float32),
            pltpu.VMEM((_L,), jnp.float32),
            pltpu.SemaphoreType.DMA,
            pltpu.SemaphoreType.DMA,
        ],
    )
    return f(inpflat, tgt, maskv)


def _tc_body(t_ref, inp_ref, mask_ref, sim_ref, o_sm_ref, buf, sems):
    i = pl.program_id(0)
    inv_tau = jnp.float32(1.0 / _TAU)

    def issue(step, slot):
        for r in range(_R):
            t = t_ref[step * _R + r]
            pltpu.make_async_copy(sim_ref.at[t], buf.at[slot, r],
                                  sems.at[slot]).start()

    @pl.when(i == 0)
    def _():
        for s in range(_NBUF - 1):
            issue(s, s)

    @pl.when(i + _NBUF - 1 < _NSTEP)
    def _():
        issue(i + _NBUF - 1, lax.rem(i + _NBUF - 1, _NBUF))

    slot = lax.rem(i, _NBUF)
    pltpu.make_async_copy(sim_ref.at[pl.ds(0, _R)], buf.at[slot],
                          sems.at[slot]).wait()

    x = buf[slot]
    e = jnp.exp(x * inv_tau)
    inp8 = inp_ref[...]
    m8 = mask_ref[...]
    s8 = jnp.sum(e, axis=1, keepdims=True)
    d8 = jnp.sum(e * inp8, axis=1, keepdims=True)
    contrib = jnp.sum(m8 * d8 / s8)

    @pl.when(i == 0)
    def _():
        o_sm_ref[0, 0] = 0.0

    o_sm_ref[0, 0] += contrib


@jax.jit
def _tc_smooth(tgt, inp2, mask2d, sim):
    return pl.pallas_call(
        _tc_body,
        grid_spec=pltpu.PrefetchScalarGridSpec(
            num_scalar_prefetch=1,
            grid=(_NSTEP,),
            in_specs=[
                pl.BlockSpec((_R, _V), lambda i, t: (i, 0)),
                pl.BlockSpec((_R, 1), lambda i, t: (i, 0)),
                pl.BlockSpec(memory_space=pltpu.MemorySpace.HBM),
            ],
            out_specs=pl.BlockSpec((1, 1), lambda i, t: (0, 0),
                                   memory_space=pltpu.MemorySpace.SMEM),
            scratch_shapes=[
                pltpu.VMEM((_NBUF, _R, _V), jnp.float32),
                pltpu.SemaphoreType.DMA((_NBUF,)),
            ],
        ),
        out_shape=jax.ShapeDtypeStruct((1, 1), jnp.float32),
        compiler_params=pltpu.CompilerParams(
            dimension_semantics=("arbitrary",)),
    )(tgt, inp2, mask2d, sim)


def kernel(input, target, mask, sim_matrix):
    inp2 = input.reshape(_N, _V)
    inpflat = input.reshape(_N * _V)
    tgt = target.reshape(_N).astype(jnp.int32)
    maskv = mask.reshape(_N)
    mask2d = mask.reshape(_N, 1)

    parts = _sc_nll(inpflat, tgt, maskv)
    smooth_sum = _tc_smooth(tgt, inp2, mask2d, sim_matrix)[0, 0]

    ml_sum = jnp.sum(parts[:, 0])
    msum = jnp.sum(parts[:, 1])
    ml_output = -ml_sum / msum
    output = _ALPHA * (-smooth_sum / msum) + (1.0 - _ALPHA) * ml_output
    return (ml_output, output)


# speedup vs baseline: 3.6249x; 1.0003x over previous
import jax
import jax.numpy as jnp
from jax import lax
from jax.experimental import pallas as pl
from jax.experimental.pallas import tpu as pltpu
from jax.experimental.pallas import tpu_sc as plsc

_B, _T, _V = 160, 16, 10000
_TAU = 0.13
_ALPHA = 0.7

_NC, _NS, _L = 2, 16, 16
_NW = _NC * _NS
_N = _B * _T
_TPW = _N // _NW

_R = 128
_NSTEP = _N // _R
_NBUF = 6


def _sc_nll_body(inpflat, tgt, maskv, out,
                 idx_v, flatidx_v, mask_v, mlvals_v, stage_v, sem_ml, sem_out):
    wid = lax.axis_index("s") * _NC + lax.axis_index("c")
    base = wid * _TPW
    iota = lax.broadcasted_iota(jnp.int32, (_L,), 0)

    pltpu.sync_copy(tgt.at[pl.ds(base, _TPW)], idx_v)
    pltpu.sync_copy(maskv.at[pl.ds(base, _TPW)], mask_v)

    for k in range(_TPW // _L):
        sl = pl.ds(k * _L, _L)
        rowid = iota + (base + k * _L)
        flatidx_v[sl] = rowid * _V + idx_v[sl]
    pltpu.async_copy(inpflat.at[flatidx_v], mlvals_v, sem_ml).wait()

    zeros = jnp.zeros((_L,), jnp.float32)
    mlacc = zeros
    msacc = zeros
    for k in range(_TPW // _L):
        sl = pl.ds(k * _L, _L)
        m16 = mask_v[sl]
        mlacc = mlacc + mlvals_v[sl] * m16
        msacc = msacc + m16
    stage = jnp.where(iota == 0, jnp.sum(mlacc), 0.0)
    stage = stage + jnp.where(iota == 1, jnp.sum(msacc), 0.0)
    stage_v[...] = stage
    pltpu.async_copy(stage_v, out.at[wid], sem_out).wait()


@jax.jit
def _sc_nll(inpflat, tgt, maskv):
    mesh = plsc.VectorSubcoreMesh(core_axis_name="c", subcore_axis_name="s",
                                  num_cores=_NC, num_subcores=_NS)
    f = pl.kernel(
        _sc_nll_body,
        out_type=jax.ShapeDtypeStruct((_NW, _L), jnp.float32),
        mesh=mesh,
        compiler_params=pltpu.CompilerParams(needs_layout_passes=False,
                                             use_tc_tiling_on_sc=False),
        scratch_types=[
            pltpu.VMEM((_TPW,), jnp.int32),
            pltpu.VMEM((_TPW,), jnp.int32),
            pltpu.VMEM((_TPW,), jnp.float32),
            pltpu.VMEM((_TPW,), jnp.float32),
            pltpu.VMEM((_L,), jnp.float32),
            pltpu.SemaphoreType.DMA,
            pltpu.SemaphoreType.DMA,
        ],
    )
    return f(inpflat, tgt, maskv)


def _tc_body(t_ref, inp_ref, mask_ref, sim_ref, o_sm_ref, buf, sems):
    i = pl.program_id(0)
    inv_tau = jnp.float32(1.0 / _TAU)

    def issue(step, slot):
        for r in range(_R):
            t = t_ref[step * _R + r]
            pltpu.make_async_copy(sim_ref.at[t], buf.at[slot, r],
                                  sems.at[slot]).start()

    @pl.when(i == 0)
    def _():
        for s in range(_NBUF - 1):
            issue(s, s)

    @pl.when(i + _NBUF - 1 < _NSTEP)
    def _():
        issue(i + _NBUF - 1, lax.rem(i + _NBUF - 1, _NBUF))

    slot = lax.rem(i, _NBUF)
    pltpu.make_async_copy(sim_ref.at[pl.ds(0, _R)], buf.at[slot],
                          sems.at[slot]).wait()

    x = buf[slot]
    e = jnp.exp(x * inv_tau)
    inp8 = inp_ref[...]
    m8 = mask_ref[...]
    s8 = jnp.sum(e, axis=1, keepdims=True)
    d8 = jnp.sum(e * inp8, axis=1, keepdims=True)
    contrib = jnp.sum(m8 * d8 / s8)

    @pl.when(i == 0)
    def _():
        o_sm_ref[0, 0] = 0.0

    o_sm_ref[0, 0] += contrib


@jax.jit
def _tc_smooth(tgt, inp2, mask2d, sim):
    return pl.pallas_call(
        _tc_body,
        grid_spec=pltpu.PrefetchScalarGridSpec(
            num_scalar_prefetch=1,
            grid=(_NSTEP,),
            in_specs=[
                pl.BlockSpec((_R, _V), lambda i, t: (i, 0)),
                pl.BlockSpec((_R, 1), lambda i, t: (i, 0)),
                pl.BlockSpec(memory_space=pltpu.MemorySpace.HBM),
            ],
            out_specs=pl.BlockSpec((1, 1), lambda i, t: (0, 0),
                                   memory_space=pltpu.MemorySpace.SMEM),
            scratch_shapes=[
                pltpu.VMEM((_NBUF, _R, _V), jnp.float32),
                pltpu.SemaphoreType.DMA((_NBUF,)),
            ],
        ),
        out_shape=jax.ShapeDtypeStruct((1, 1), jnp.float32),
        compiler_params=pltpu.CompilerParams(
            dimension_semantics=("arbitrary",)),
    )(tgt, inp2, mask2d, sim)


def kernel(input, target, mask, sim_matrix):
    inp2 = input.reshape(_N, _V)
    inpflat = input.reshape(_N * _V)
    tgt = target.reshape(_N).astype(jnp.int32)
    maskv = mask.reshape(_N)
    mask2d = mask.reshape(_N, 1)

    parts = _sc_nll(inpflat, tgt, maskv)
    smooth_sum = _tc_smooth(tgt, inp2, mask2d, sim_matrix)[0, 0]

    ml_sum = jnp.sum(parts[:, 0])
    msum = jnp.sum(parts[:, 1])
    ml_output = -ml_sum / msum
    output = _ALPHA * (-smooth_sum / msum) + (1.0 - _ALPHA) * ml_output
    return (ml_output, output)
